# Initial kernel scaffold; baseline (speedup 1.0000x reference)
import jax
import jax.numpy as jnp
from jax.experimental import pallas as pl


def kernel(x, edge_index, batch, W1, b1, W2, b2, W3, b3):
    raise NotImplementedError("write your pallas kernel here")


# speedup vs baseline: 30.0034x; 30.0034x over previous
import functools

import jax
import jax.numpy as jnp
from jax import lax
from jax.experimental import pallas as pl
from jax.experimental.pallas import tpu as pltpu
from jax.experimental.pallas import tpu_sc as plsc

N = 10000
NP = 10240
D = 50
NE = 640000
NC = 2
NS = 16
NW = NC * NS
EPT = NE // NW
CK = 80
CH = EPT // CK
RPT = NP // NS
G = 256
GB = 384
NPT = NP // NW
PCH = NPT // CK

_mesh = functools.partial(
    plsc.VectorSubcoreMesh, core_axis_name="c", subcore_axis_name="s")

_SC_PARAMS = pltpu.CompilerParams(use_tc_tiling_on_sc=False)


def _f32(*shape):
    return jax.ShapeDtypeStruct(shape, jnp.float32)


def _deg_cnt_body(dst_hbm, batch_hbm, zn_hbm, deg_hbm, cnt_hbm,
                  didx_v, bidx_v, ones_v, dacc_sh, cacc_sh):
    c = lax.axis_index("c")
    s = lax.axis_index("s")
    wid = c * NS + s
    for i in range(CK // 16):
        ones_v[pl.ds(16 * i, 16)] = jnp.ones((16,), jnp.float32)
    pltpu.sync_copy(zn_hbm.at[pl.ds(s * RPT, RPT)], dacc_sh.at[pl.ds(s * RPT, RPT)])

    @pl.when(s == 0)
    def _():
        pltpu.sync_copy(zn_hbm.at[pl.ds(0, GB)], cacc_sh)

    pltpu.sync_copy(dst_hbm.at[wid], didx_v)
    pltpu.sync_copy(batch_hbm.at[wid], bidx_v)
    plsc.subcore_barrier()

    def ebody(j, carry):
        pltpu.sync_copy(ones_v, dacc_sh.at[didx_v.at[j]], add=True)
        return carry

    lax.fori_loop(0, CH, ebody, 0)

    def bbody(j, carry):
        pltpu.sync_copy(ones_v, cacc_sh.at[bidx_v.at[j]], add=True)
        return carry

    lax.fori_loop(0, PCH, bbody, 0)
    plsc.subcore_barrier()
    pltpu.sync_copy(dacc_sh.at[pl.ds(s * RPT, RPT)], deg_hbm.at[c, pl.ds(s * RPT, RPT)])

    @pl.when(s == 0)
    def _():
        pltpu.sync_copy(cacc_sh, cnt_hbm.at[c])


def _deg_cnt(dst_r, batch_r, zn):
    return pl.kernel(
        _deg_cnt_body,
        out_type=(_f32(NC, NP), _f32(NC, GB)),
        mesh=_mesh(),
        compiler_params=_SC_PARAMS,
        scratch_types=[
            pltpu.VMEM((CH, CK), jnp.int32),
            pltpu.VMEM((PCH, CK), jnp.int32),
            pltpu.VMEM((CK,), jnp.float32),
            pltpu.VMEM_SHARED((NP,), jnp.float32),
            pltpu.VMEM_SHARED((GB,), jnp.float32),
        ],
    )(dst_r, batch_r, zn)


def _agg_body(src_hbm, dst_hbm, g_hbm, zn2_hbm, out_hbm,
              sidx_v, didx_v, rows_v, acc_sh, sem):
    c = lax.axis_index("c")
    s = lax.axis_index("s")
    wid = c * NS + s
    pltpu.sync_copy(zn2_hbm.at[pl.ds(s * RPT, RPT), :],
                    acc_sh.at[pl.ds(s * RPT, RPT), :])
    pltpu.sync_copy(src_hbm.at[wid], sidx_v)
    pltpu.sync_copy(dst_hbm.at[wid], didx_v)
    plsc.subcore_barrier()

    def body(j, carry):
        pltpu.async_copy(g_hbm.at[sidx_v.at[j]], rows_v, sem).wait()
        pltpu.sync_copy(rows_v, acc_sh.at[didx_v.at[j]], add=True)
        return carry

    lax.fori_loop(0, CH, body, 0)
    plsc.subcore_barrier()
    pltpu.sync_copy(acc_sh.at[pl.ds(s * RPT, RPT), :],
                    out_hbm.at[c, pl.ds(s * RPT, RPT), :])


def _aggregate(src_r, dst_r, g, zn2):
    return pl.kernel(
        _agg_body,
        out_type=_f32(NC, NP, D),
        mesh=_mesh(),
        compiler_params=_SC_PARAMS,
        scratch_types=[
            pltpu.VMEM((CH, CK), jnp.int32),
            pltpu.VMEM((CH, CK), jnp.int32),
            pltpu.VMEM((CK, D), jnp.float32),
            pltpu.VMEM_SHARED((NP, D), jnp.float32),
            pltpu.SemaphoreType.DMA,
        ],
    )(src_r, dst_r, g, zn2)


def _pool_body(h_hbm, batch_hbm, zb_hbm, out_hbm, bidx_v, rows_v, acc_sh):
    c = lax.axis_index("c")
    s = lax.axis_index("s")
    wid = c * NS + s

    @pl.when(s == 0)
    def _():
        pltpu.sync_copy(zb_hbm, acc_sh)

    pltpu.sync_copy(batch_hbm.at[wid], bidx_v)
    plsc.subcore_barrier()

    def body(j, carry):
        pltpu.sync_copy(h_hbm.at[pl.ds(wid * NPT + j * CK, CK), :], rows_v)
        pltpu.sync_copy(rows_v, acc_sh.at[bidx_v.at[j]], add=True)
        return carry

    lax.fori_loop(0, PCH, body, 0)
    plsc.subcore_barrier()

    @pl.when(s == 0)
    def _():
        pltpu.sync_copy(acc_sh, out_hbm.at[c])


def _pool(h, batch_r, zb):
    return pl.kernel(
        _pool_body,
        out_type=_f32(NC, GB, D),
        mesh=_mesh(),
        compiler_params=_SC_PARAMS,
        scratch_types=[
            pltpu.VMEM((PCH, CK), jnp.int32),
            pltpu.VMEM((CK, D), jnp.float32),
            pltpu.VMEM_SHARED((GB, D), jnp.float32),
        ],
    )(h, batch_r, zb)


def _row_mask():
    return (lax.broadcasted_iota(jnp.int32, (NP, 1), 0) < N).astype(jnp.float32)


def _scale_in_body(x_ref, w_ref, degp_ref, g_ref):
    dinv = lax.rsqrt(degp_ref[0] + degp_ref[1] + 2.0)
    h = jnp.dot(x_ref[...], w_ref[...], preferred_element_type=jnp.float32,
                 precision=lax.Precision.HIGHEST)
    g_ref[0:N, :] = h * dinv[:N, None]
    g_ref[N:NP, :] = jnp.zeros((NP - N, D), jnp.float32)


def _scale_in(x, W1, degp):
    return pl.pallas_call(_scale_in_body, out_shape=_f32(NP, D))(x, W1, degp)


def _mid_body(sp_ref, g_ref, degp_ref, b_ref, w_ref, out_ref):
    dinv = lax.rsqrt(degp_ref[0] + degp_ref[1] + 2.0)[:, None]
    sfull = sp_ref[0] + sp_ref[1]
    o = jnp.maximum(dinv * (sfull + 2.0 * g_ref[...]) + b_ref[...][None, :], 0.0)
    o = o * _row_mask()
    h2 = jnp.dot(o, w_ref[...], preferred_element_type=jnp.float32,
                 precision=lax.Precision.HIGHEST)
    out_ref[...] = h2 * dinv


def _mid(s1p, g1, degp, b1, W2):
    return pl.pallas_call(_mid_body, out_shape=_f32(NP, D))(s1p, g1, degp, b1, W2)


def _out_body(sp_ref, g_ref, degp_ref, b_ref, out_ref):
    dinv = lax.rsqrt(degp_ref[0] + degp_ref[1] + 2.0)[:, None]
    sfull = sp_ref[0] + sp_ref[1]
    o = jnp.maximum(dinv * (sfull + 2.0 * g_ref[...]) + b_ref[...][None, :], 0.0)
    out_ref[...] = o * _row_mask()


def _outl(s2p, g2, degp, b2):
    return pl.pallas_call(_out_body, out_shape=_f32(NP, D))(s2p, g2, degp, b2)


def _head_body(sums_ref, cnt_ref, w3_ref, b3_ref, out_ref):
    sums = sums_ref[0, 0:G, :] + sums_ref[1, 0:G, :]
    cnt = cnt_ref[0, 0:G] + cnt_ref[1, 0:G]
    pooled = sums / jnp.maximum(cnt, 1.0)[:, None]
    z = jnp.dot(pooled, w3_ref[...], preferred_element_type=jnp.float32,
                 precision=lax.Precision.HIGHEST)
    out_ref[...] = jax.nn.sigmoid(z + b3_ref[...][None, :])


def _head(sumsp, cntp, W3, b3):
    return pl.pallas_call(_head_body, out_shape=_f32(G, 1))(sumsp, cntp, W3, b3)


def kernel(x, edge_index, batch, W1, b1, W2, b2, W3, b3):
    src_r = edge_index[0].reshape(NW, CH, CK)
    dst_r = edge_index[1].reshape(NW, CH, CK)
    batch_r = jnp.concatenate(
        [batch, jnp.full((NP - N,), G, jnp.int32)]).reshape(NW, PCH, CK)
    zn = jnp.zeros((NP,), jnp.float32)
    zn2 = jnp.zeros((NP, D), jnp.float32)
    zb = jnp.zeros((GB, D), jnp.float32)

    degp, cntp = _deg_cnt(dst_r, batch_r, zn)
    g1 = _scale_in(x, W1, degp)
    s1p = _aggregate(src_r, dst_r, g1, zn2)
    g2 = _mid(s1p, g1, degp, b1, W2)
    s2p = _aggregate(src_r, dst_r, g2, zn2)
    h2 = _outl(s2p, g2, degp, b2)
    sumsp = _pool(h2, batch_r, zb)
    out = _head(sumsp, cntp, W3, b3)
    return out.reshape(G)
